# baseline (speedup 1.0000x reference)
import functools

import jax
import jax.numpy as jnp
from jax import lax
from jax.experimental import pallas as pl
from jax.experimental.pallas import tpu as pltpu
from jax.experimental.pallas import tpu_sc as plsc

NC = 2
NS = 16


def _sc_segment_sum(table, src, dst, zeros, *, n_pad, n_edges, d):
    nw = NC * NS
    epw = n_edges // nw
    k = 80
    nch = epw // k
    nb = 4
    ni = 8
    rpt = n_pad // NS
    mesh = plsc.VectorSubcoreMesh(core_axis_name="c", subcore_axis_name="s")

    @functools.partial(
        pl.kernel,
        out_type=jax.ShapeDtypeStruct((NC * n_pad, d), jnp.float32),
        mesh=mesh,
        scratch_types=[
            [pltpu.VMEM((k,), jnp.int32) for _ in range(ni)],
            [pltpu.VMEM((k,), jnp.int32) for _ in range(ni)],
            [pltpu.VMEM((k, d), jnp.float32) for _ in range(nb)],
            pltpu.VMEM_SHARED((n_pad, d), jnp.float32),
            [pltpu.SemaphoreType.DMA for _ in range(ni)],
            [pltpu.SemaphoreType.DMA for _ in range(nb)],
            [pltpu.SemaphoreType.DMA for _ in range(nb)],
        ],
    )
    def agg(table_h, src_h, dst_h, zeros_h, out_h,
            srcb, dstb, rows, acc, semi, semg, sems):
        cid = lax.axis_index("c")
        sid = lax.axis_index("s")
        wid = sid * NC + cid
        base = wid * epw
        r0 = sid * rpt

        def fire_idx(i, q):
            b = base + i * k
            pltpu.async_copy(src_h.at[pl.ds(b, k)], srcb[q], semi[q])
            pltpu.async_copy(dst_h.at[pl.ds(b, k)], dstb[q], semi[q])

        def wait_idx(q):
            pltpu.make_async_copy(src_h.at[pl.ds(base, k)], srcb[q], semi[q]).wait()
            pltpu.make_async_copy(dst_h.at[pl.ds(base, k)], dstb[q], semi[q]).wait()

        def fire_gather(q, p):
            pltpu.async_copy(table_h.at[srcb[q]], rows[p], semg[p])

        def wait_gather(q, p):
            pltpu.make_async_copy(table_h.at[srcb[q]], rows[p], semg[p]).wait()

        def fire_scatter(q, p):
            pltpu.async_copy(rows[p], acc.at[dstb[q]], sems[p], add=True)

        def wait_scatter(q, p):
            pltpu.make_async_copy(rows[p], acc.at[dstb[q]], sems[p]).wait()

        for j in range(min(5, nch)):
            fire_idx(j, j)
        for j in range(min(3, nch)):
            wait_idx(j)
            fire_gather(j, j)
        pltpu.sync_copy(zeros_h.at[pl.ds(r0, rpt)], acc.at[pl.ds(r0, rpt)])
        plsc.subcore_barrier()

        def step(i, p, q):
            wait_gather(q, p)
            fire_scatter(q, p)

            @pl.when(i >= 1)
            def _drain_scatter():
                wait_scatter((q - 1) % ni, (p - 1) % nb)

            @pl.when(i + 3 < nch)
            def _fire_next_gather():
                wait_idx((q + 3) % ni)
                fire_gather((q + 3) % ni, (p + 3) % nb)

            @pl.when(i + 5 < nch)
            def _prefetch_idx():
                fire_idx(i + 5, (q + 5) % ni)

        def octet(j, carry):
            for u in range(ni):
                i = ni * j + u

                @pl.when(i < nch)
                def _chunk():
                    step(i, u % nb, u)

            return carry

        lax.fori_loop(0, (nch + ni - 1) // ni, octet, 0)
        if nch >= 1:
            wait_scatter((nch - 1) % ni, (nch - 1) % nb)
        plsc.subcore_barrier()
        pltpu.sync_copy(acc.at[pl.ds(r0, rpt)],
                        out_h.at[pl.ds(cid * n_pad + r0, rpt)])

    return agg(table, src, dst, zeros)


def _tc_mm1(x, W1, *, n_nodes):
    br = 2000
    in_ch = W1.shape[0]
    hid = W1.shape[1]

    def body(x_ref, w1_ref, o_ref):
        o_ref[...] = jnp.dot(x_ref[...], w1_ref[...],
                             preferred_element_type=jnp.float32)

    return pl.pallas_call(
        body,
        grid=(n_nodes // br,),
        in_specs=[
            pl.BlockSpec((br, in_ch), lambda i: (i, 0)),
            pl.BlockSpec((in_ch, hid), lambda i: (0, 0)),
        ],
        out_specs=pl.BlockSpec((br, hid), lambda i: (i, 0)),
        out_shape=jax.ShapeDtypeStruct((n_nodes, hid), jnp.float32),
    )(x, W1)


def _tc_elu_mm2(p, b1, W2, *, n_nodes):
    br = 2000
    hid = W2.shape[0]
    out_ch = W2.shape[1]

    def body(p_ref, b1_ref, w2_ref, o_ref):
        h = p_ref[0] + p_ref[1] + b1_ref[...]
        h = jnp.where(h > 0, h, jnp.exp(jnp.minimum(h, 0.0)) - 1.0)
        t = jnp.dot(h, w2_ref[...], preferred_element_type=jnp.float32)
        o_ref[...] = jnp.pad(t, ((0, 0), (0, hid - out_ch)))

    return pl.pallas_call(
        body,
        grid=(n_nodes // br,),
        in_specs=[
            pl.BlockSpec((NC, br, hid), lambda i: (0, i, 0)),
            pl.BlockSpec((1, hid), lambda i: (0, 0)),
            pl.BlockSpec((hid, out_ch), lambda i: (0, 0)),
        ],
        out_specs=pl.BlockSpec((br, hid), lambda i: (i, 0)),
        out_shape=jax.ShapeDtypeStruct((n_nodes, hid), jnp.float32),
    )(p, b1, W2)


def _tc_softmax(q, b2, *, n_nodes, out_ch):
    br = 2000
    hid = q.shape[-1]

    def body(q_ref, b2_ref, o_ref):
        t = q_ref[0, :, :out_ch] + q_ref[1, :, :out_ch] + b2_ref[...]
        m = jnp.max(t, axis=1, keepdims=True)
        e = jnp.exp(t - m)
        o_ref[...] = e / jnp.sum(e, axis=1, keepdims=True)

    return pl.pallas_call(
        body,
        grid=(n_nodes // br,),
        in_specs=[
            pl.BlockSpec((NC, br, hid), lambda i: (0, i, 0)),
            pl.BlockSpec((1, out_ch), lambda i: (0, 0)),
        ],
        out_specs=pl.BlockSpec((br, out_ch), lambda i: (i, 0)),
        out_shape=jax.ShapeDtypeStruct((n_nodes, out_ch), jnp.float32),
    )(q, b2)


def kernel(x, edge_index, W1, b1, W2, b2):
    n, in_ch = x.shape
    n_edges = edge_index.shape[1]
    hid = W1.shape[1]
    out_ch = W2.shape[1]

    npad = -(-n // (NS * 8)) * (NS * 8)
    src = edge_index[0]
    dst = edge_index[1]

    h1 = _tc_mm1(x, W1, n_nodes=n)
    p1 = _sc_segment_sum(h1, src, dst, jnp.zeros((npad, hid), jnp.float32),
                         n_pad=npad, n_edges=n_edges, d=hid)
    p1 = p1.reshape(NC, npad, hid)

    h3 = _tc_elu_mm2(p1, b1.reshape(1, hid), W2, n_nodes=n)
    p2 = _sc_segment_sum(h3, src, dst, jnp.zeros((npad, hid), jnp.float32),
                         n_pad=npad, n_edges=n_edges, d=hid)
    p2 = p2.reshape(NC, npad, hid)
    return _tc_softmax(p2, b2.reshape(1, out_ch), n_nodes=n, out_ch=out_ch)


# speedup vs baseline: 1.0617x; 1.0617x over previous
import functools

import jax
import jax.numpy as jnp
from jax import lax
from jax.experimental import pallas as pl
from jax.experimental.pallas import tpu as pltpu
from jax.experimental.pallas import tpu_sc as plsc

NC = 2
NS = 16


def _sc_segment_sum(table, src, dst, zeros, *, n_pad, n_edges, d):
    nw = NC * NS
    epw = n_edges // nw
    k = 80
    nch = epw // k
    nb = 4
    ni = 8
    rpt = n_pad // NS
    mesh = plsc.VectorSubcoreMesh(core_axis_name="c", subcore_axis_name="s")

    @functools.partial(
        pl.kernel,
        out_type=jax.ShapeDtypeStruct((NC * n_pad, d), jnp.float32),
        mesh=mesh,
        scratch_types=[
            [pltpu.VMEM((k,), jnp.int32) for _ in range(ni)],
            [pltpu.VMEM((k,), jnp.int32) for _ in range(ni)],
            [pltpu.VMEM((k, d), jnp.float32) for _ in range(nb)],
            pltpu.VMEM_SHARED((n_pad, d), jnp.float32),
            [pltpu.SemaphoreType.DMA for _ in range(ni)],
            [pltpu.SemaphoreType.DMA for _ in range(nb)],
            [pltpu.SemaphoreType.DMA for _ in range(nb)],
        ],
    )
    def agg(table_h, src_h, dst_h, zeros_h, out_h,
            srcb, dstb, rows, acc, semi, semg, sems):
        cid = lax.axis_index("c")
        sid = lax.axis_index("s")
        wid = sid * NC + cid
        base = wid * epw
        r0 = sid * rpt

        def fire_idx(i, q):
            b = base + i * k
            pltpu.async_copy(src_h.at[pl.ds(b, k)], srcb[q], semi[q])
            pltpu.async_copy(dst_h.at[pl.ds(b, k)], dstb[q], semi[q])

        def wait_idx(q):
            pltpu.make_async_copy(src_h.at[pl.ds(base, k)], srcb[q], semi[q]).wait()
            pltpu.make_async_copy(dst_h.at[pl.ds(base, k)], dstb[q], semi[q]).wait()

        def fire_gather(q, p):
            pltpu.async_copy(table_h.at[srcb[q]], rows[p], semg[p])

        def wait_gather(q, p):
            pltpu.make_async_copy(table_h.at[srcb[q]], rows[p], semg[p]).wait()

        def fire_scatter(q, p):
            pltpu.async_copy(rows[p], acc.at[dstb[q]], sems[p], add=True)

        def wait_scatter(q, p):
            pltpu.make_async_copy(rows[p], acc.at[dstb[q]], sems[p]).wait()

        for j in range(min(5, nch)):
            fire_idx(j, j)
        pltpu.sync_copy(zeros_h.at[pl.ds(r0, rpt)], acc.at[pl.ds(r0, rpt)])
        plsc.subcore_barrier()
        for j in range(min(3, nch)):
            wait_idx(j)
            fire_gather(j, j)

        def step(i, p, q):
            @pl.when(i >= 1)
            def _drain_scatter():
                wait_scatter((q - 1) % ni, (p - 1) % nb)

            @pl.when(i + 3 < nch)
            def _fire_next_gather():
                wait_idx((q + 3) % ni)
                fire_gather((q + 3) % ni, (p + 3) % nb)

            wait_gather(q, p)
            fire_scatter(q, p)

            @pl.when(i + 5 < nch)
            def _prefetch_idx():
                fire_idx(i + 5, (q + 5) % ni)

        def octet(j, carry):
            for u in range(ni):
                i = ni * j + u

                @pl.when(i < nch)
                def _chunk():
                    step(i, u % nb, u)

            return carry

        lax.fori_loop(0, (nch + ni - 1) // ni, octet, 0)
        if nch >= 1:
            wait_scatter((nch - 1) % ni, (nch - 1) % nb)
        plsc.subcore_barrier()
        pltpu.sync_copy(acc.at[pl.ds(r0, rpt)],
                        out_h.at[pl.ds(cid * n_pad + r0, rpt)])

    return agg(table, src, dst, zeros)


def _tc_mm1(x, W1, *, n_nodes):
    br = 2000
    in_ch = W1.shape[0]
    hid = W1.shape[1]

    def body(x_ref, w1_ref, o_ref):
        o_ref[...] = jnp.dot(x_ref[...], w1_ref[...],
                             preferred_element_type=jnp.float32)

    return pl.pallas_call(
        body,
        grid=(n_nodes // br,),
        in_specs=[
            pl.BlockSpec((br, in_ch), lambda i: (i, 0)),
            pl.BlockSpec((in_ch, hid), lambda i: (0, 0)),
        ],
        out_specs=pl.BlockSpec((br, hid), lambda i: (i, 0)),
        out_shape=jax.ShapeDtypeStruct((n_nodes, hid), jnp.float32),
    )(x, W1)


def _tc_elu_mm2(p, b1, W2, *, n_nodes):
    br = 2000
    hid = W2.shape[0]
    out_ch = W2.shape[1]

    def body(p_ref, b1_ref, w2_ref, o_ref):
        h = p_ref[0] + p_ref[1] + b1_ref[...]
        h = jnp.where(h > 0, h, jnp.exp(jnp.minimum(h, 0.0)) - 1.0)
        t = jnp.dot(h, w2_ref[...], preferred_element_type=jnp.float32)
        o_ref[...] = jnp.pad(t, ((0, 0), (0, hid - out_ch)))

    return pl.pallas_call(
        body,
        grid=(n_nodes // br,),
        in_specs=[
            pl.BlockSpec((NC, br, hid), lambda i: (0, i, 0)),
            pl.BlockSpec((1, hid), lambda i: (0, 0)),
            pl.BlockSpec((hid, out_ch), lambda i: (0, 0)),
        ],
        out_specs=pl.BlockSpec((br, hid), lambda i: (i, 0)),
        out_shape=jax.ShapeDtypeStruct((n_nodes, hid), jnp.float32),
    )(p, b1, W2)


def _tc_softmax(q, b2, *, n_nodes, out_ch):
    br = 2000
    hid = q.shape[-1]

    def body(q_ref, b2_ref, o_ref):
        t = q_ref[0, :, :out_ch] + q_ref[1, :, :out_ch] + b2_ref[...]
        m = jnp.max(t, axis=1, keepdims=True)
        e = jnp.exp(t - m)
        o_ref[...] = e / jnp.sum(e, axis=1, keepdims=True)

    return pl.pallas_call(
        body,
        grid=(n_nodes // br,),
        in_specs=[
            pl.BlockSpec((NC, br, hid), lambda i: (0, i, 0)),
            pl.BlockSpec((1, out_ch), lambda i: (0, 0)),
        ],
        out_specs=pl.BlockSpec((br, out_ch), lambda i: (i, 0)),
        out_shape=jax.ShapeDtypeStruct((n_nodes, out_ch), jnp.float32),
    )(q, b2)


def kernel(x, edge_index, W1, b1, W2, b2):
    n, in_ch = x.shape
    n_edges = edge_index.shape[1]
    hid = W1.shape[1]
    out_ch = W2.shape[1]

    npad = -(-n // (NS * 8)) * (NS * 8)
    src = edge_index[0]
    dst = edge_index[1]

    h1 = _tc_mm1(x, W1, n_nodes=n)
    p1 = _sc_segment_sum(h1, src, dst, jnp.zeros((npad, hid), jnp.float32),
                         n_pad=npad, n_edges=n_edges, d=hid)
    p1 = p1.reshape(NC, npad, hid)

    h3 = _tc_elu_mm2(p1, b1.reshape(1, hid), W2, n_nodes=n)
    p2 = _sc_segment_sum(h3, src, dst, jnp.zeros((npad, hid), jnp.float32),
                         n_pad=npad, n_edges=n_edges, d=hid)
    p2 = p2.reshape(NC, npad, hid)
    return _tc_softmax(p2, b2.reshape(1, out_ch), n_nodes=n, out_ch=out_ch)
